# baseline (speedup 1.0000x reference)
import functools

import jax
import jax.numpy as jnp
from jax import lax
from jax.experimental import pallas as pl
from jax.experimental.pallas import tpu as pltpu
from jax.experimental.pallas import tpu_sc as plsc

N = 10000
NP = 10240
E = 160000
EP = 163840
D = 256
DH = 128
HID = 512
NB = 16
BLK = NP // NB
CH = 80
NCH = EP // 16 // CH
GRP = 16
NGRP = NCH // GRP
RING = 4
EH = EP // 32

_mesh = plsc.VectorSubcoreMesh(core_axis_name="c", subcore_axis_name="s")
_sc_params = pltpu.CompilerParams(needs_layout_passes=False)


def _hist_body(dst_hbm, hist_out, dst_v, hist_v):
    c = lax.axis_index("c")
    s = lax.axis_index("s")
    wid = s * 2 + c
    pltpu.sync_copy(dst_hbm.at[wid], dst_v)

    def zero(i, _):
        hist_v[pl.ds(i * 16, 16)] = jnp.zeros((16,), jnp.float32)
        return 0

    lax.fori_loop(0, NP // 16, zero, 0)
    ones = jnp.ones((16,), jnp.float32)

    def add(i, _):
        idx = dst_v[pl.ds(i * 16, 16)]
        plsc.addupdate_scatter(hist_v, [idx], ones)
        return 0

    lax.fori_loop(0, EH // 16, add, 0)
    pltpu.sync_copy(hist_v, hist_out.at[wid])


_hist_call = functools.partial(
    pl.kernel,
    out_type=jax.ShapeDtypeStruct((32, NP), jnp.float32),
    mesh=_mesh,
    scratch_types=[
        pltpu.VMEM((EH,), jnp.int32),
        pltpu.VMEM((NP,), jnp.float32),
    ],
    compiler_params=_sc_params,
)(_hist_body)


def _scale_body(x_ref, hist_ref, xs2_ref, dis8_ref, disc_ref):
    deg = jnp.sum(hist_ref[...], axis=0, keepdims=True) + 1.0
    dis_row = lax.rsqrt(deg)
    dis_col = jnp.transpose(dis_row)
    xv = x_ref[...]
    xc = jnp.where(jnp.isnan(xv), 0.0,
                   jnp.where(xv == jnp.inf, 1.0,
                             jnp.where(xv == -jnp.inf, -1.0, xv)))
    xs = dis_col * xc
    xs2_ref[0] = xs[:, :DH]
    xs2_ref[1] = xs[:, DH:]
    dis8_ref[...] = jnp.broadcast_to(dis_row, (8, BLK))
    disc_ref[...] = jnp.broadcast_to(dis_col, (BLK, 8))


_scale_call = pl.pallas_call(
    _scale_body,
    grid=(NB,),
    in_specs=[
        pl.BlockSpec((BLK, D), lambda i: (i, 0)),
        pl.BlockSpec((32, BLK), lambda i: (0, i)),
    ],
    out_specs=[
        pl.BlockSpec((2, BLK, DH), lambda i: (0, i, 0)),
        pl.BlockSpec((8, BLK), lambda i: (0, i)),
        pl.BlockSpec((BLK, 8), lambda i: (i, 0)),
    ],
    out_shape=[
        jax.ShapeDtypeStruct((2, NP, DH), jnp.float32),
        jax.ShapeDtypeStruct((8, NP), jnp.float32),
        jax.ShapeDtypeStruct((NP, 8), jnp.float32),
    ],
)


def _agg_body(xs2, src_t, dst_t, dis8,
              agg2, t16,
              src_v, dst_v, rows0, rows1, dis_v, t_v, acc,
              gsem0, gsem1, ssem0, ssem1):
    c = lax.axis_index("c")
    s = lax.axis_index("s")
    rows = (rows0, rows1)
    gsem = (gsem0, gsem1)
    ssem = (ssem0, ssem1)
    xs = xs2.at[c]
    pltpu.sync_copy(dis8.at[0], dis_v)

    def zrows(i, _):
        rows0[i // 8, pl.ds((i % 8) * 16, 16)] = jnp.zeros((16,), jnp.float32)
        return 0

    lax.fori_loop(0, CH * 8, zrows, 0)
    for k in range(BLK // CH):
        pltpu.sync_copy(rows0, acc.at[pl.ds(s * BLK + k * CH, CH)])

    def zt(i, _):
        t_v[pl.ds(i * 16, 16)] = jnp.zeros((16,), jnp.float32)
        return 0

    lax.fori_loop(0, NP // 16, zt, 0)
    plsc.subcore_barrier()

    def group(g, _):
        pltpu.sync_copy(src_t.at[s, pl.ds(g * GRP, GRP)], src_v)
        pltpu.sync_copy(dst_t.at[s, pl.ds(g * GRP, GRP)], dst_v)
        gd = {0: pltpu.async_copy(xs.at[src_v.at[0]], rows0, gsem0)}
        sd = {}
        for jj in range(GRP):
            b = jj % 2
            gd[jj].wait()
            if jj + 1 < GRP:
                if jj - 1 >= 0:
                    sd[jj - 1].wait()
                gd[jj + 1] = pltpu.async_copy(
                    xs.at[src_v.at[jj + 1]], rows[1 - b], gsem[1 - b])

            @pl.when(c == 0)
            def _(jj=jj):
                def tstep(l, _):
                    dsti = dst_v[jj, pl.ds(l * 16, 16)]
                    srci = src_v[jj, pl.ds(l * 16, 16)]
                    dvals = plsc.load_gather(dis_v, [dsti])
                    plsc.addupdate_scatter(t_v, [srci], dvals)
                    return 0

                lax.fori_loop(0, CH // 16, tstep, 0)

            sd[jj] = pltpu.async_copy(
                rows[b], acc.at[dst_v.at[jj]], ssem[b], add=True)
        sd[GRP - 2].wait()
        sd[GRP - 1].wait()
        return 0

    lax.fori_loop(0, NGRP, group, 0)

    plsc.subcore_barrier()
    out = agg2.at[c]
    for k in range(BLK // CH):
        pltpu.sync_copy(acc.at[pl.ds(s * BLK + k * CH, CH)],
                        out.at[pl.ds(s * BLK + k * CH, CH)])

    @pl.when(c == 0)
    def _():
        pltpu.sync_copy(t_v, t16.at[s])


_agg_call = functools.partial(
    pl.kernel,
    out_type=(
        jax.ShapeDtypeStruct((2, NP, DH), jnp.float32),
        jax.ShapeDtypeStruct((16, NP), jnp.float32),
    ),
    mesh=_mesh,
    scratch_types=[
        pltpu.VMEM((GRP, CH), jnp.int32),
        pltpu.VMEM((GRP, CH), jnp.int32),
        pltpu.VMEM((CH, DH), jnp.float32),
        pltpu.VMEM((CH, DH), jnp.float32),
        pltpu.VMEM((NP,), jnp.float32),
        pltpu.VMEM((NP,), jnp.float32),
        pltpu.VMEM_SHARED((NP, DH), jnp.float32),
        pltpu.SemaphoreType.DMA,
        pltpu.SemaphoreType.DMA,
        pltpu.SemaphoreType.DMA,
        pltpu.SemaphoreType.DMA,
    ],
    compiler_params=_sc_params,
)(_agg_body)


def _final_body(agg2_ref, xs2_ref, disc_ref, dis8_ref,
                t32_ref, w1_ref, b1_ref, w2_ref, b2_ref, out_ref, s_ref):
    i = pl.program_id(0)
    dis_col = disc_ref[:, 0:1]
    y = dis_col * jnp.concatenate(
        [agg2_ref[0] + xs2_ref[0], agg2_ref[1] + xs2_ref[1]], axis=1)
    h1 = lax.dot_general(y, w1_ref[...], (((1,), (1,)), ((), ())),
                         preferred_element_type=jnp.float32)
    h1 = jnp.maximum(h1 + b1_ref[...], 0.0)

    dis_row = dis8_ref[0:1, :]
    t_row = jnp.sum(t32_ref[...], axis=0, keepdims=True)
    rowid = lax.broadcasted_iota(jnp.int32, (1, BLK), 1) + i * BLK
    w = jnp.where(rowid < N, dis_row * t_row + dis_row * dis_row, 0.0)
    part = lax.dot_general(w, h1, (((1,), (0,)), ((), ())),
                           preferred_element_type=jnp.float32)

    @pl.when(i == 0)
    def _():
        s_ref[...] = jnp.zeros_like(s_ref)

    s_ref[0:1, :] += part

    @pl.when(i == NB - 1)
    def _():
        sfin = s_ref[0:1, :] * (1.0 / N)
        o = lax.dot_general(sfin, w2_ref[...], (((1,), (1,)), ((), ())),
                            preferred_element_type=jnp.float32)
        out_ref[...] = o + b2_ref[...]


_final_call = pl.pallas_call(
    _final_body,
    grid=(NB,),
    in_specs=[
        pl.BlockSpec((2, BLK, DH), lambda i: (0, i, 0)),
        pl.BlockSpec((2, BLK, DH), lambda i: (0, i, 0)),
        pl.BlockSpec((BLK, 8), lambda i: (i, 0)),
        pl.BlockSpec((8, BLK), lambda i: (0, i)),
        pl.BlockSpec((16, BLK), lambda i: (0, i)),
        pl.BlockSpec((HID, D), lambda i: (0, 0)),
        pl.BlockSpec((1, HID), lambda i: (0, 0)),
        pl.BlockSpec((8, HID), lambda i: (0, 0)),
        pl.BlockSpec((1, 8), lambda i: (0, 0)),
    ],
    out_specs=pl.BlockSpec((1, 8), lambda i: (0, 0)),
    out_shape=jax.ShapeDtypeStruct((1, 8), jnp.float32),
    scratch_shapes=[pltpu.VMEM((8, HID), jnp.float32)],
)


def kernel(x, edge_index, W1, b1, W2, b2):
    src = edge_index[0].astype(jnp.int32)
    dst = edge_index[1].astype(jnp.int32)
    pad = jnp.full((EP - E,), N, jnp.int32)
    srcp = jnp.concatenate([src, pad])
    dstp = jnp.concatenate([dst, pad])
    dst_h = dstp.reshape(32, EH)
    src_t = srcp.reshape(16, NCH, CH)
    dst_t = dstp.reshape(16, NCH, CH)

    hist = _hist_call(dst_h)
    xs2, dis8, disc = _scale_call(x, hist)
    agg2, t16 = _agg_call(xs2, src_t, dst_t, dis8)

    w2p = jnp.zeros((8, HID), jnp.float32).at[:3].set(W2)
    b2p = jnp.zeros((1, 8), jnp.float32).at[0, :3].set(b2)
    out = _final_call(agg2, xs2, disc, dis8, t16,
                      W1, b1.reshape(1, HID), w2p, b2p)
    return out[:, :3]


# speedup vs baseline: 1.2253x; 1.2253x over previous
import functools

import jax
import jax.numpy as jnp
from jax import lax
from jax.experimental import pallas as pl
from jax.experimental.pallas import tpu as pltpu
from jax.experimental.pallas import tpu_sc as plsc

N = 10000
NP = 10240
E = 160000
EP = 163840
D = 256
DH = 128
HID = 512
NB = 16
BLK = NP // NB
CH = 80
NCH = EP // 16 // CH
GRP = 16
NGRP = NCH // GRP
RING = 4
EH = EP // 32

_mesh = plsc.VectorSubcoreMesh(core_axis_name="c", subcore_axis_name="s")
_sc_params = pltpu.CompilerParams(needs_layout_passes=False)


def _hist_body(dst_hbm, hist_out, dst_v, hist_v):
    c = lax.axis_index("c")
    s = lax.axis_index("s")
    wid = s * 2 + c
    pltpu.sync_copy(dst_hbm.at[wid], dst_v)

    def zero(i, _):
        hist_v[pl.ds(i * 16, 16)] = jnp.zeros((16,), jnp.float32)
        return 0

    lax.fori_loop(0, NP // 16, zero, 0)
    ones = jnp.ones((16,), jnp.float32)

    def add(i, _):
        idx = dst_v[pl.ds(i * 16, 16)]
        plsc.addupdate_scatter(hist_v, [idx], ones)
        return 0

    lax.fori_loop(0, EH // 16, add, 0)
    pltpu.sync_copy(hist_v, hist_out.at[wid])


_hist_call = functools.partial(
    pl.kernel,
    out_type=jax.ShapeDtypeStruct((32, NP), jnp.float32),
    mesh=_mesh,
    scratch_types=[
        pltpu.VMEM((EH,), jnp.int32),
        pltpu.VMEM((NP,), jnp.float32),
    ],
    compiler_params=_sc_params,
)(_hist_body)


def _scale_body(x_ref, hist_ref, xs2_ref, dis8_ref, disc_ref):
    deg = jnp.sum(hist_ref[...], axis=0, keepdims=True) + 1.0
    dis_row = lax.rsqrt(deg)
    dis_col = jnp.transpose(dis_row)
    xv = x_ref[...]
    xc = jnp.where(jnp.isnan(xv), 0.0,
                   jnp.where(xv == jnp.inf, 1.0,
                             jnp.where(xv == -jnp.inf, -1.0, xv)))
    xs = dis_col * xc
    xs2_ref[0] = xs[:, :DH]
    xs2_ref[1] = xs[:, DH:]
    dis8_ref[...] = jnp.broadcast_to(dis_row, (8, BLK))
    disc_ref[...] = jnp.broadcast_to(dis_col, (BLK, 8))


_scale_call = pl.pallas_call(
    _scale_body,
    grid=(NB,),
    in_specs=[
        pl.BlockSpec((BLK, D), lambda i: (i, 0)),
        pl.BlockSpec((32, BLK), lambda i: (0, i)),
    ],
    out_specs=[
        pl.BlockSpec((2, BLK, DH), lambda i: (0, i, 0)),
        pl.BlockSpec((8, BLK), lambda i: (0, i)),
        pl.BlockSpec((BLK, 8), lambda i: (i, 0)),
    ],
    out_shape=[
        jax.ShapeDtypeStruct((2, NP, DH), jnp.float32),
        jax.ShapeDtypeStruct((8, NP), jnp.float32),
        jax.ShapeDtypeStruct((NP, 8), jnp.float32),
    ],
)


def _t_body(src_hbm, dst_hbm, dis8, t_out, src_v, dst_v, dis_v, t_v):
    c = lax.axis_index("c")
    s = lax.axis_index("s")
    wid = s * 2 + c
    pltpu.sync_copy(src_hbm.at[wid], src_v)
    pltpu.sync_copy(dst_hbm.at[wid], dst_v)
    pltpu.sync_copy(dis8.at[0], dis_v)

    def zt(i, _):
        t_v[pl.ds(i * 16, 16)] = jnp.zeros((16,), jnp.float32)
        return 0

    lax.fori_loop(0, NP // 16, zt, 0)

    def tstep(l, _):
        dsti = dst_v[pl.ds(l * 16, 16)]
        srci = src_v[pl.ds(l * 16, 16)]
        dvals = plsc.load_gather(dis_v, [dsti])
        plsc.addupdate_scatter(t_v, [srci], dvals)
        return 0

    lax.fori_loop(0, EH // 16, tstep, 0)
    pltpu.sync_copy(t_v, t_out.at[wid])


_t_call = functools.partial(
    pl.kernel,
    out_type=jax.ShapeDtypeStruct((32, NP), jnp.float32),
    mesh=_mesh,
    scratch_types=[
        pltpu.VMEM((EH,), jnp.int32),
        pltpu.VMEM((EH,), jnp.int32),
        pltpu.VMEM((NP,), jnp.float32),
        pltpu.VMEM((NP,), jnp.float32),
    ],
    compiler_params=_sc_params,
)(_t_body)


def _agg_body(xs2, src_t, dst_t,
              agg2,
              src_v, dst_v, rows0, rows1, rows2, rows3, acc,
              gsem0, gsem1, gsem2, gsem3, ssem0, ssem1, ssem2, ssem3):
    c = lax.axis_index("c")
    s = lax.axis_index("s")
    rows = (rows0, rows1, rows2, rows3)
    gsem = (gsem0, gsem1, gsem2, gsem3)
    ssem = (ssem0, ssem1, ssem2, ssem3)
    xs = xs2.at[c]

    def zrows(i, _):
        rows0[i // 8, pl.ds((i % 8) * 16, 16)] = jnp.zeros((16,), jnp.float32)
        return 0

    lax.fori_loop(0, CH * 8, zrows, 0)
    for k in range(BLK // CH):
        pltpu.sync_copy(rows0, acc.at[pl.ds(s * BLK + k * CH, CH)])
    plsc.subcore_barrier()

    def group(g, _):
        pltpu.sync_copy(src_t.at[s, pl.ds(g * GRP, GRP)], src_v)
        pltpu.sync_copy(dst_t.at[s, pl.ds(g * GRP, GRP)], dst_v)
        gd = {}
        sd = {}
        for jj in range(min(2, GRP)):
            gd[jj] = pltpu.async_copy(
                xs.at[src_v.at[jj]], rows[jj % RING], gsem[jj % RING])
        for jj in range(GRP):
            b = jj % RING
            gd[jj].wait()
            sd[jj] = pltpu.async_copy(
                rows[b], acc.at[dst_v.at[jj]], ssem[b], add=True)
            if jj + 2 < GRP:
                nb = (jj + 2) % RING
                if jj - 2 >= 0:
                    sd[jj - 2].wait()
                gd[jj + 2] = pltpu.async_copy(
                    xs.at[src_v.at[jj + 2]], rows[nb], gsem[nb])
        for jj in range(max(0, GRP - RING), GRP):
            sd[jj].wait()
        return 0

    lax.fori_loop(0, NGRP, group, 0)

    plsc.subcore_barrier()
    out = agg2.at[c]
    for k in range(BLK // CH):
        pltpu.sync_copy(acc.at[pl.ds(s * BLK + k * CH, CH)],
                        out.at[pl.ds(s * BLK + k * CH, CH)])


_agg_call = functools.partial(
    pl.kernel,
    out_type=jax.ShapeDtypeStruct((2, NP, DH), jnp.float32),
    mesh=_mesh,
    scratch_types=[
        pltpu.VMEM((GRP, CH), jnp.int32),
        pltpu.VMEM((GRP, CH), jnp.int32),
        pltpu.VMEM((CH, DH), jnp.float32),
        pltpu.VMEM((CH, DH), jnp.float32),
        pltpu.VMEM((CH, DH), jnp.float32),
        pltpu.VMEM((CH, DH), jnp.float32),
        pltpu.VMEM_SHARED((NP, DH), jnp.float32),
        pltpu.SemaphoreType.DMA,
        pltpu.SemaphoreType.DMA,
        pltpu.SemaphoreType.DMA,
        pltpu.SemaphoreType.DMA,
        pltpu.SemaphoreType.DMA,
        pltpu.SemaphoreType.DMA,
        pltpu.SemaphoreType.DMA,
        pltpu.SemaphoreType.DMA,
    ],
    compiler_params=_sc_params,
)(_agg_body)


def _final_body(agg2_ref, xs2_ref, disc_ref, dis8_ref,
                t32_ref, w1_ref, b1_ref, w2_ref, b2_ref, out_ref, s_ref):
    i = pl.program_id(0)
    dis_col = disc_ref[:, 0:1]
    y = dis_col * jnp.concatenate(
        [agg2_ref[0] + xs2_ref[0], agg2_ref[1] + xs2_ref[1]], axis=1)
    h1 = lax.dot_general(y, w1_ref[...], (((1,), (1,)), ((), ())),
                         preferred_element_type=jnp.float32)
    h1 = jnp.maximum(h1 + b1_ref[...], 0.0)

    dis_row = dis8_ref[0:1, :]
    t_row = jnp.sum(t32_ref[...], axis=0, keepdims=True)
    rowid = lax.broadcasted_iota(jnp.int32, (1, BLK), 1) + i * BLK
    w = jnp.where(rowid < N, dis_row * t_row + dis_row * dis_row, 0.0)
    part = lax.dot_general(w, h1, (((1,), (0,)), ((), ())),
                           preferred_element_type=jnp.float32)

    @pl.when(i == 0)
    def _():
        s_ref[...] = jnp.zeros_like(s_ref)

    s_ref[0:1, :] += part

    @pl.when(i == NB - 1)
    def _():
        sfin = s_ref[0:1, :] * (1.0 / N)
        o = lax.dot_general(sfin, w2_ref[...], (((1,), (1,)), ((), ())),
                            preferred_element_type=jnp.float32)
        out_ref[...] = o + b2_ref[...]


_final_call = pl.pallas_call(
    _final_body,
    grid=(NB,),
    in_specs=[
        pl.BlockSpec((2, BLK, DH), lambda i: (0, i, 0)),
        pl.BlockSpec((2, BLK, DH), lambda i: (0, i, 0)),
        pl.BlockSpec((BLK, 8), lambda i: (i, 0)),
        pl.BlockSpec((8, BLK), lambda i: (0, i)),
        pl.BlockSpec((32, BLK), lambda i: (0, i)),
        pl.BlockSpec((HID, D), lambda i: (0, 0)),
        pl.BlockSpec((1, HID), lambda i: (0, 0)),
        pl.BlockSpec((8, HID), lambda i: (0, 0)),
        pl.BlockSpec((1, 8), lambda i: (0, 0)),
    ],
    out_specs=pl.BlockSpec((1, 8), lambda i: (0, 0)),
    out_shape=jax.ShapeDtypeStruct((1, 8), jnp.float32),
    scratch_shapes=[pltpu.VMEM((8, HID), jnp.float32)],
)


def kernel(x, edge_index, W1, b1, W2, b2):
    src = edge_index[0].astype(jnp.int32)
    dst = edge_index[1].astype(jnp.int32)
    pad = jnp.full((EP - E,), N, jnp.int32)
    srcp = jnp.concatenate([src, pad])
    dstp = jnp.concatenate([dst, pad])
    src_h = srcp.reshape(32, EH)
    dst_h = dstp.reshape(32, EH)
    src_t = srcp.reshape(16, NCH, CH)
    dst_t = dstp.reshape(16, NCH, CH)

    hist = _hist_call(dst_h)
    xs2, dis8, disc = _scale_call(x, hist)
    t32 = _t_call(src_h, dst_h, dis8)
    agg2 = _agg_call(xs2, src_t, dst_t)

    w2p = jnp.zeros((8, HID), jnp.float32).at[:3].set(W2)
    b2p = jnp.zeros((1, 8), jnp.float32).at[0, :3].set(b2)
    out = _final_call(agg2, xs2, disc, dis8, t32,
                      W1, b1.reshape(1, HID), w2p, b2p)
    return out[:, :3]


# speedup vs baseline: 1.9123x; 1.5606x over previous
import functools

import jax
import jax.numpy as jnp
from jax import lax
from jax.experimental import pallas as pl
from jax.experimental.pallas import tpu as pltpu
from jax.experimental.pallas import tpu_sc as plsc

N = 10000
NP = 10240
E = 160000
D = 256
DH = 128
HID = 512
NB = 16
BLK = NP // NB
CH = 80
NCH = E // 16 // CH
GRP = 5
NGRP = NCH // GRP
RING = 4
EP = 163840
EH = EP // 32
EH16 = EH // 16

_mesh = plsc.VectorSubcoreMesh(core_axis_name="c", subcore_axis_name="s")
_sc_params = pltpu.CompilerParams(needs_layout_passes=False)


def _hist_body(dst_hbm, hist_out, dst_v, hist_v):
    c = lax.axis_index("c")
    s = lax.axis_index("s")
    wid = s * 2 + c
    pltpu.sync_copy(dst_hbm.at[wid], dst_v)

    def zero(i, _):
        hist_v[pl.ds(i * 16, 16)] = jnp.zeros((16,), jnp.float32)
        return 0

    lax.fori_loop(0, NP // 16, zero, 0)
    ones = jnp.ones((16,), jnp.float32)

    def add(i, _):
        idx = dst_v[pl.ds(i * 16, 16)]
        plsc.addupdate_scatter(hist_v, [idx], ones)
        return 0

    lax.fori_loop(0, EH16, add, 0)
    pltpu.sync_copy(hist_v, hist_out.at[wid])


_hist_call = functools.partial(
    pl.kernel,
    out_type=jax.ShapeDtypeStruct((32, NP), jnp.float32),
    mesh=_mesh,
    scratch_types=[
        pltpu.VMEM((EH,), jnp.int32),
        pltpu.VMEM((NP,), jnp.float32),
    ],
    compiler_params=_sc_params,
)(_hist_body)


def _scale_body(x_ref, hist_ref, xs2_ref, dis8_ref, disc_ref):
    deg = jnp.sum(hist_ref[...], axis=0, keepdims=True) + 1.0
    dis_row = lax.rsqrt(deg)
    dis_col = jnp.transpose(dis_row)
    xv = x_ref[...]
    xc = jnp.where(jnp.isnan(xv), 0.0,
                   jnp.where(xv == jnp.inf, 1.0,
                             jnp.where(xv == -jnp.inf, -1.0, xv)))
    xs = dis_col * xc
    xs2_ref[0] = xs[:, :DH]
    xs2_ref[1] = xs[:, DH:]
    dis8_ref[...] = jnp.broadcast_to(dis_row, (8, BLK))
    disc_ref[...] = jnp.broadcast_to(dis_col, (BLK, 8))


_scale_call = pl.pallas_call(
    _scale_body,
    grid=(NB,),
    in_specs=[
        pl.BlockSpec((BLK, D), lambda i: (i, 0)),
        pl.BlockSpec((32, BLK), lambda i: (0, i)),
    ],
    out_specs=[
        pl.BlockSpec((2, BLK, DH), lambda i: (0, i, 0)),
        pl.BlockSpec((8, BLK), lambda i: (0, i)),
        pl.BlockSpec((BLK, 8), lambda i: (i, 0)),
    ],
    out_shape=[
        jax.ShapeDtypeStruct((2, NP, DH), jnp.float32),
        jax.ShapeDtypeStruct((8, NP), jnp.float32),
        jax.ShapeDtypeStruct((NP, 8), jnp.float32),
    ],
)


def _t_body(src_hbm, dst_hbm, dis8, t_out, src_v, dst_v, dis_v, t_v):
    c = lax.axis_index("c")
    s = lax.axis_index("s")
    wid = s * 2 + c
    pltpu.sync_copy(src_hbm.at[wid], src_v.at[pl.ds(0, EH)])
    pltpu.sync_copy(dst_hbm.at[wid], dst_v)
    pltpu.sync_copy(dis8.at[0], dis_v)

    def zt(i, _):
        t_v[pl.ds(i * 16, 16)] = jnp.zeros((16,), jnp.float32)
        return 0

    lax.fori_loop(0, NP // 16, zt, 0)

    def tstep(l, _):
        dsti = dst_v[pl.ds(l * 16, 16)]
        srci = src_v[pl.ds(l * 16, 16)]
        dvals = plsc.load_gather(dis_v, [dsti])
        plsc.addupdate_scatter(t_v, [srci], dvals)
        return 0

    lax.fori_loop(0, EH16, tstep, 0)
    pltpu.sync_copy(t_v, t_out.at[wid])


_t_call = functools.partial(
    pl.kernel,
    out_type=jax.ShapeDtypeStruct((32, NP), jnp.float32),
    mesh=_mesh,
    scratch_types=[
        pltpu.VMEM((EH,), jnp.int32),
        pltpu.VMEM((EH,), jnp.int32),
        pltpu.VMEM((NP,), jnp.float32),
        pltpu.VMEM((NP,), jnp.float32),
    ],
    compiler_params=_sc_params,
)(_t_body)


def _agg_body(xs2, src_t, dst_t,
              agg2,
              src_v, dst_v, rows0, rows1, rows2, rows3, acc,
              gsem0, gsem1, gsem2, gsem3, ssem0, ssem1, ssem2, ssem3):
    c = lax.axis_index("c")
    s = lax.axis_index("s")
    rows = (rows0, rows1, rows2, rows3)
    gsem = (gsem0, gsem1, gsem2, gsem3)
    ssem = (ssem0, ssem1, ssem2, ssem3)
    xs = xs2.at[c]

    def zrows(i, _):
        rows0[i // 8, pl.ds((i % 8) * 16, 16)] = jnp.zeros((16,), jnp.float32)
        return 0

    lax.fori_loop(0, CH * 8, zrows, 0)
    for k in range(BLK // CH):
        pltpu.sync_copy(rows0, acc.at[pl.ds(s * BLK + k * CH, CH)])
    plsc.subcore_barrier()

    def group(g, _):
        pltpu.sync_copy(src_t.at[s, g], src_v)
        pltpu.sync_copy(dst_t.at[s, g], dst_v)
        gd = {}
        sd = {}
        for jj in range(min(2, GRP)):
            gd[jj] = pltpu.async_copy(
                xs.at[src_v.at[jj]], rows[jj % RING], gsem[jj % RING])
        for jj in range(GRP):
            b = jj % RING
            gd[jj].wait()
            sd[jj] = pltpu.async_copy(
                rows[b], acc.at[dst_v.at[jj]], ssem[b], add=True)
            if jj + 2 < GRP:
                nb = (jj + 2) % RING
                if jj - 2 >= 0:
                    sd[jj - 2].wait()
                gd[jj + 2] = pltpu.async_copy(
                    xs.at[src_v.at[jj + 2]], rows[nb], gsem[nb])
        for jj in range(max(0, GRP - RING), GRP):
            sd[jj].wait()
        return 0

    lax.fori_loop(0, NGRP, group, 0)

    plsc.subcore_barrier()
    out = agg2.at[c]
    for k in range(BLK // CH):
        pltpu.sync_copy(acc.at[pl.ds(s * BLK + k * CH, CH)],
                        out.at[pl.ds(s * BLK + k * CH, CH)])


_agg_call = functools.partial(
    pl.kernel,
    out_type=jax.ShapeDtypeStruct((2, NP, DH), jnp.float32),
    mesh=_mesh,
    scratch_types=[
        pltpu.VMEM((GRP, CH), jnp.int32),
        pltpu.VMEM((GRP, CH), jnp.int32),
        pltpu.VMEM((CH, DH), jnp.float32),
        pltpu.VMEM((CH, DH), jnp.float32),
        pltpu.VMEM((CH, DH), jnp.float32),
        pltpu.VMEM((CH, DH), jnp.float32),
        pltpu.VMEM_SHARED((NP, DH), jnp.float32),
        pltpu.SemaphoreType.DMA,
        pltpu.SemaphoreType.DMA,
        pltpu.SemaphoreType.DMA,
        pltpu.SemaphoreType.DMA,
        pltpu.SemaphoreType.DMA,
        pltpu.SemaphoreType.DMA,
        pltpu.SemaphoreType.DMA,
        pltpu.SemaphoreType.DMA,
    ],
    compiler_params=_sc_params,
)(_agg_body)


def _final_body(agg2_ref, xs2_ref, disc_ref, dis8_ref,
                t32_ref, w1_ref, b1_ref, w2_ref, b2_ref, out_ref, s_ref):
    i = pl.program_id(0)
    dis_col = disc_ref[:, 0:1]
    y = dis_col * jnp.concatenate(
        [agg2_ref[0] + xs2_ref[0], agg2_ref[1] + xs2_ref[1]], axis=1)
    h1 = lax.dot_general(y, w1_ref[...], (((1,), (1,)), ((), ())),
                         preferred_element_type=jnp.float32)
    h1 = jnp.maximum(h1 + b1_ref[...], 0.0)

    dis_row = dis8_ref[0:1, :]
    t_row = jnp.sum(t32_ref[...], axis=0, keepdims=True)
    rowid = lax.broadcasted_iota(jnp.int32, (1, BLK), 1) + i * BLK
    w = jnp.where(rowid < N, dis_row * t_row + dis_row * dis_row, 0.0)
    part = lax.dot_general(w, h1, (((1,), (0,)), ((), ())),
                           preferred_element_type=jnp.float32)

    @pl.when(i == 0)
    def _():
        s_ref[...] = jnp.zeros_like(s_ref)

    s_ref[0:1, :] += part

    @pl.when(i == NB - 1)
    def _():
        sfin = s_ref[0:1, :] * (1.0 / N)
        o = lax.dot_general(sfin, w2_ref[...], (((1,), (1,)), ((), ())),
                            preferred_element_type=jnp.float32)
        out_ref[...] = o + b2_ref[...]


_final_call = pl.pallas_call(
    _final_body,
    grid=(NB,),
    in_specs=[
        pl.BlockSpec((2, BLK, DH), lambda i: (0, i, 0)),
        pl.BlockSpec((2, BLK, DH), lambda i: (0, i, 0)),
        pl.BlockSpec((BLK, 8), lambda i: (i, 0)),
        pl.BlockSpec((8, BLK), lambda i: (0, i)),
        pl.BlockSpec((32, BLK), lambda i: (0, i)),
        pl.BlockSpec((HID, D), lambda i: (0, 0)),
        pl.BlockSpec((1, HID), lambda i: (0, 0)),
        pl.BlockSpec((3, HID), lambda i: (0, 0)),
        pl.BlockSpec((1, 3), lambda i: (0, 0)),
    ],
    out_specs=pl.BlockSpec((1, 3), lambda i: (0, 0)),
    out_shape=jax.ShapeDtypeStruct((1, 3), jnp.float32),
    scratch_shapes=[pltpu.VMEM((8, HID), jnp.float32)],
)


def kernel(x, edge_index, W1, b1, W2, b2):
    src = edge_index[0].astype(jnp.int32)
    dst = edge_index[1].astype(jnp.int32)
    pad = jnp.full((EP - E,), N, jnp.int32)
    src_h = jnp.concatenate([src, pad]).reshape(32, EH)
    dst_h = jnp.concatenate([dst, pad]).reshape(32, EH)
    src_t = src.reshape(16, NGRP, GRP, CH)
    dst_t = dst.reshape(16, NGRP, GRP, CH)

    hist = _hist_call(dst_h)
    xs2, dis8, disc = _scale_call(x, hist)
    t32 = _t_call(src_h, dst_h, dis8)
    agg2 = _agg_call(xs2, src_t, dst_t)

    out = _final_call(agg2, xs2, disc, dis8, t32,
                      W1, b1.reshape(1, HID), W2, b2.reshape(1, 3))
    return out


# speedup vs baseline: 2.2177x; 1.1597x over previous
import functools

import jax
import jax.numpy as jnp
from jax import lax
from jax.experimental import pallas as pl
from jax.experimental.pallas import tpu as pltpu
from jax.experimental.pallas import tpu_sc as plsc

N = 10000
NP = 10240
E = 160000
D = 256
DH = 128
HID = 512
NB = 16
BLK = NP // NB
CH = 80
NCH = E // 16 // CH
GRP = 25
NGRP = NCH // GRP
RING = 4
EP = 163840
EH = EP // 32
EH16 = EH // 16

_mesh = plsc.VectorSubcoreMesh(core_axis_name="c", subcore_axis_name="s")
_sc_params = pltpu.CompilerParams(needs_layout_passes=False)


def _hist_body(dst_hbm, hist_out, dst_v, hist_v):
    c = lax.axis_index("c")
    s = lax.axis_index("s")
    wid = s * 2 + c
    pltpu.sync_copy(dst_hbm.at[wid], dst_v)

    def zero(i, _):
        hist_v[pl.ds(i * 16, 16)] = jnp.zeros((16,), jnp.float32)
        return 0

    lax.fori_loop(0, NP // 16, zero, 0)
    ones = jnp.ones((16,), jnp.float32)

    def add(i, _):
        idx = dst_v[pl.ds(i * 16, 16)]
        plsc.addupdate_scatter(hist_v, [idx], ones)
        return 0

    lax.fori_loop(0, EH16, add, 0)
    pltpu.sync_copy(hist_v, hist_out.at[wid])


_hist_call = functools.partial(
    pl.kernel,
    out_type=jax.ShapeDtypeStruct((32, NP), jnp.float32),
    mesh=_mesh,
    scratch_types=[
        pltpu.VMEM((EH,), jnp.int32),
        pltpu.VMEM((NP,), jnp.float32),
    ],
    compiler_params=_sc_params,
)(_hist_body)


def _scale_body(x_ref, hist_ref, xs2_ref, dis8_ref, disc_ref):
    deg = jnp.sum(hist_ref[...], axis=0, keepdims=True) + 1.0
    dis_row = lax.rsqrt(deg)
    dis_col = jnp.transpose(dis_row)
    xv = x_ref[...]
    xc = jnp.where(jnp.isnan(xv), 0.0,
                   jnp.where(xv == jnp.inf, 1.0,
                             jnp.where(xv == -jnp.inf, -1.0, xv)))
    xs = dis_col * xc
    xs2_ref[0] = xs[:, :DH]
    xs2_ref[1] = xs[:, DH:]
    dis8_ref[...] = jnp.broadcast_to(dis_row, (8, BLK))
    disc_ref[...] = jnp.broadcast_to(dis_col, (BLK, 8))


_scale_call = pl.pallas_call(
    _scale_body,
    grid=(NB,),
    in_specs=[
        pl.BlockSpec((BLK, D), lambda i: (i, 0)),
        pl.BlockSpec((32, BLK), lambda i: (0, i)),
    ],
    out_specs=[
        pl.BlockSpec((2, BLK, DH), lambda i: (0, i, 0)),
        pl.BlockSpec((8, BLK), lambda i: (0, i)),
        pl.BlockSpec((BLK, 8), lambda i: (i, 0)),
    ],
    out_shape=[
        jax.ShapeDtypeStruct((2, NP, DH), jnp.float32),
        jax.ShapeDtypeStruct((8, NP), jnp.float32),
        jax.ShapeDtypeStruct((NP, 8), jnp.float32),
    ],
)


def _t_body(src_hbm, dst_hbm, dis8, t_out, src_v, dst_v, dis_v, t_v):
    c = lax.axis_index("c")
    s = lax.axis_index("s")
    wid = s * 2 + c
    pltpu.sync_copy(src_hbm.at[wid], src_v.at[pl.ds(0, EH)])
    pltpu.sync_copy(dst_hbm.at[wid], dst_v)
    pltpu.sync_copy(dis8.at[0], dis_v)

    def zt(i, _):
        t_v[pl.ds(i * 16, 16)] = jnp.zeros((16,), jnp.float32)
        return 0

    lax.fori_loop(0, NP // 16, zt, 0)

    def tstep(l, _):
        dsti = dst_v[pl.ds(l * 16, 16)]
        srci = src_v[pl.ds(l * 16, 16)]
        dvals = plsc.load_gather(dis_v, [dsti])
        plsc.addupdate_scatter(t_v, [srci], dvals)
        return 0

    lax.fori_loop(0, EH16, tstep, 0)
    pltpu.sync_copy(t_v, t_out.at[wid])


_t_call = functools.partial(
    pl.kernel,
    out_type=jax.ShapeDtypeStruct((32, NP), jnp.float32),
    mesh=_mesh,
    scratch_types=[
        pltpu.VMEM((EH,), jnp.int32),
        pltpu.VMEM((EH,), jnp.int32),
        pltpu.VMEM((NP,), jnp.float32),
        pltpu.VMEM((NP,), jnp.float32),
    ],
    compiler_params=_sc_params,
)(_t_body)


def _agg_body(xs2, src_t, dst_t,
              agg2,
              src_v, dst_v, rows0, rows1, rows2, rows3, acc,
              gsem0, gsem1, gsem2, gsem3, ssem0, ssem1, ssem2, ssem3):
    c = lax.axis_index("c")
    s = lax.axis_index("s")
    rows = (rows0, rows1, rows2, rows3)
    gsem = (gsem0, gsem1, gsem2, gsem3)
    ssem = (ssem0, ssem1, ssem2, ssem3)
    xs = xs2.at[c]

    def zrows(i, _):
        rows0[i // 8, pl.ds((i % 8) * 16, 16)] = jnp.zeros((16,), jnp.float32)
        return 0

    lax.fori_loop(0, CH * 8, zrows, 0)
    for k in range(BLK // CH):
        pltpu.sync_copy(rows0, acc.at[pl.ds(s * BLK + k * CH, CH)])
    plsc.subcore_barrier()

    def group(g, _):
        pltpu.sync_copy(src_t.at[s, g], src_v)
        pltpu.sync_copy(dst_t.at[s, g], dst_v)
        gd = {}
        sd = {}
        for jj in range(min(2, GRP)):
            gd[jj] = pltpu.async_copy(
                xs.at[src_v.at[jj]], rows[jj % RING], gsem[jj % RING])
        for jj in range(GRP):
            b = jj % RING
            gd[jj].wait()
            sd[jj] = pltpu.async_copy(
                rows[b], acc.at[dst_v.at[jj]], ssem[b], add=True)
            if jj + 2 < GRP:
                nb = (jj + 2) % RING
                if jj - 2 >= 0:
                    sd[jj - 2].wait()
                gd[jj + 2] = pltpu.async_copy(
                    xs.at[src_v.at[jj + 2]], rows[nb], gsem[nb])
        for jj in range(max(0, GRP - RING), GRP):
            sd[jj].wait()
        return 0

    lax.fori_loop(0, NGRP, group, 0)

    plsc.subcore_barrier()
    out = agg2.at[c]
    for k in range(BLK // CH):
        pltpu.sync_copy(acc.at[pl.ds(s * BLK + k * CH, CH)],
                        out.at[pl.ds(s * BLK + k * CH, CH)])


_agg_call = functools.partial(
    pl.kernel,
    out_type=jax.ShapeDtypeStruct((2, NP, DH), jnp.float32),
    mesh=_mesh,
    scratch_types=[
        pltpu.VMEM((GRP, CH), jnp.int32),
        pltpu.VMEM((GRP, CH), jnp.int32),
        pltpu.VMEM((CH, DH), jnp.float32),
        pltpu.VMEM((CH, DH), jnp.float32),
        pltpu.VMEM((CH, DH), jnp.float32),
        pltpu.VMEM((CH, DH), jnp.float32),
        pltpu.VMEM_SHARED((NP, DH), jnp.float32),
        pltpu.SemaphoreType.DMA,
        pltpu.SemaphoreType.DMA,
        pltpu.SemaphoreType.DMA,
        pltpu.SemaphoreType.DMA,
        pltpu.SemaphoreType.DMA,
        pltpu.SemaphoreType.DMA,
        pltpu.SemaphoreType.DMA,
        pltpu.SemaphoreType.DMA,
    ],
    compiler_params=_sc_params,
)(_agg_body)


def _final_body(agg2_ref, xs2_ref, disc_ref, dis8_ref,
                t32_ref, w1_ref, b1_ref, w2_ref, b2_ref, out_ref, s_ref):
    i = pl.program_id(0)
    dis_col = disc_ref[:, 0:1]
    y = dis_col * jnp.concatenate(
        [agg2_ref[0] + xs2_ref[0], agg2_ref[1] + xs2_ref[1]], axis=1)
    h1 = lax.dot_general(y, w1_ref[...], (((1,), (1,)), ((), ())),
                         preferred_element_type=jnp.float32)
    h1 = jnp.maximum(h1 + b1_ref[...], 0.0)

    dis_row = dis8_ref[0:1, :]
    t_row = jnp.sum(t32_ref[...], axis=0, keepdims=True)
    rowid = lax.broadcasted_iota(jnp.int32, (1, BLK), 1) + i * BLK
    w = jnp.where(rowid < N, dis_row * t_row + dis_row * dis_row, 0.0)
    part = lax.dot_general(w, h1, (((1,), (0,)), ((), ())),
                           preferred_element_type=jnp.float32)

    @pl.when(i == 0)
    def _():
        s_ref[...] = jnp.zeros_like(s_ref)

    s_ref[0:1, :] += part

    @pl.when(i == NB - 1)
    def _():
        sfin = s_ref[0:1, :] * (1.0 / N)
        o = lax.dot_general(sfin, w2_ref[...], (((1,), (1,)), ((), ())),
                            preferred_element_type=jnp.float32)
        out_ref[...] = o + b2_ref[...]


_final_call = pl.pallas_call(
    _final_body,
    grid=(NB,),
    in_specs=[
        pl.BlockSpec((2, BLK, DH), lambda i: (0, i, 0)),
        pl.BlockSpec((2, BLK, DH), lambda i: (0, i, 0)),
        pl.BlockSpec((BLK, 8), lambda i: (i, 0)),
        pl.BlockSpec((8, BLK), lambda i: (0, i)),
        pl.BlockSpec((32, BLK), lambda i: (0, i)),
        pl.BlockSpec((HID, D), lambda i: (0, 0)),
        pl.BlockSpec((1, HID), lambda i: (0, 0)),
        pl.BlockSpec((3, HID), lambda i: (0, 0)),
        pl.BlockSpec((1, 3), lambda i: (0, 0)),
    ],
    out_specs=pl.BlockSpec((1, 3), lambda i: (0, 0)),
    out_shape=jax.ShapeDtypeStruct((1, 3), jnp.float32),
    scratch_shapes=[pltpu.VMEM((8, HID), jnp.float32)],
)


def kernel(x, edge_index, W1, b1, W2, b2):
    src = edge_index[0].astype(jnp.int32)
    dst = edge_index[1].astype(jnp.int32)
    pad = jnp.full((EP - E,), N, jnp.int32)
    src_h = jnp.concatenate([src, pad]).reshape(32, EH)
    dst_h = jnp.concatenate([dst, pad]).reshape(32, EH)
    src_t = src.reshape(16, NGRP, GRP, CH)
    dst_t = dst.reshape(16, NGRP, GRP, CH)

    hist = _hist_call(dst_h)
    xs2, dis8, disc = _scale_call(x, hist)
    t32 = _t_call(src_h, dst_h, dis8)
    agg2 = _agg_call(xs2, src_t, dst_t)

    out = _final_call(agg2, xs2, disc, dis8, t32,
                      W1, b1.reshape(1, HID), W2, b2.reshape(1, 3))
    return out
